# baseline (speedup 1.0000x reference)
import functools

import jax
import jax.numpy as jnp
from jax import lax
from jax.experimental import pallas as pl
from jax.experimental.pallas import tpu as pltpu
from jax.experimental.pallas import tpu_sc as plsc

EMBED_DIM = 64
W = 64
NB = 4
NSLOT = 2 * NB


def _sc_gather(table, idx, n):
    ncores, nsub = 2, 16
    nw = ncores * nsub
    per_w = n // nw
    steps = per_w // W
    assert per_w % W == 0 and steps % NSLOT == 0

    mesh = plsc.VectorSubcoreMesh(core_axis_name="core", subcore_axis_name="subcore")

    @functools.partial(
        pl.kernel,
        out_type=jax.ShapeDtypeStruct((n, EMBED_DIM), table.dtype),
        mesh=mesh,
        compiler_params=pltpu.CompilerParams(use_tc_tiling_on_sc=False),
        scratch_types=(
            [pltpu.VMEM((NSLOT, W), jnp.int32),
             pltpu.VMEM((NSLOT, W, EMBED_DIM), jnp.float32)]
            + [pltpu.SemaphoreType.DMA] * (3 * NSLOT)
        ),
    )
    def k(table_hbm, idx_hbm, out_hbm, idx_v, rows_v, *sems):
        isem = sems[0:NSLOT]
        gsem = sems[NSLOT:2 * NSLOT]
        ssem = sems[2 * NSLOT:3 * NSLOT]
        wid = lax.axis_index("subcore") * ncores + lax.axis_index("core")
        base = wid * per_w

        def idx_dma(slot, s):
            return pltpu.make_async_copy(
                idx_hbm.at[pl.ds(base + s * W, W)], idx_v.at[slot], isem[slot])

        def gather_dma(slot):
            return pltpu.make_async_copy(
                table_hbm.at[idx_v.at[slot]], rows_v.at[slot], gsem[slot])

        def store_dma(slot, s):
            return pltpu.make_async_copy(
                rows_v.at[slot], out_hbm.at[pl.ds(base + s * W, W)], ssem[slot])

        for b in range(NSLOT):
            idx_dma(b, b).start()

        @pl.loop(0, steps, step=NSLOT)
        def _(r0):
            for b in range(NSLOT):
                s = r0 + b
                @pl.when(s >= NSLOT)
                def _():
                    store_dma(b, s - NSLOT).wait()
                idx_dma(b, s).wait()
                gather_dma(b).start()
                bl = (b - NB) % NSLOT

                @pl.when(s >= NB)
                def _():
                    gather_dma(bl).wait()
                    store_dma(bl, s - NB).start()

                @pl.when(jnp.logical_and(s >= NB, s + NB < steps))
                def _():
                    idx_dma(bl, s + NB).start()

        for i in range(NB):
            s = steps - NB + i
            b = s % NSLOT
            gather_dma(b).wait()
            store_dma(b, s).start()
        for i in range(NSLOT):
            s = steps - NSLOT + i
            b = s % NSLOT
            store_dma(b, s).wait()

    return k(table, idx)


def kernel(x, table):
    b, h = x.shape
    n = b * h
    idx = x.reshape(n).astype(jnp.int32)
    out = _sc_gather(table, idx, n)
    return out.reshape(b, h, EMBED_DIM)


# speedup vs baseline: 2.7198x; 2.7198x over previous
import jax
import jax.numpy as jnp
from jax.experimental import pallas as pl
from jax.experimental.pallas import tpu as pltpu
from jax.experimental.pallas import tpu_sc as plsc

EMBED_DIM = 64
WINDOW = 32


def _sc_gather(table, idx, n):
    mesh = plsc.VectorSubcoreMesh(core_axis_name="core", subcore_axis_name="subcore")

    @pl.kernel(
        out_type=jax.ShapeDtypeStruct((n, EMBED_DIM), table.dtype),
        mesh=mesh,
        compiler_params=pltpu.CompilerParams(use_tc_tiling_on_sc=False),
        scratch_types=[pltpu.VMEM_SHARED((1000, EMBED_DIM), jnp.float32)],
    )
    def k(table_hbm, idx_hbm, out_hbm, table_sh):
        from jax import lax

        @pl.when(lax.axis_index("subcore") == 0)
        def _():
            pltpu.sync_copy(table_hbm, table_sh)

        plsc.subcore_barrier()

        def body(i_vmem, o_vmem):
            pltpu.sync_copy(table_sh.at[i_vmem.at[0]], o_vmem)

        pltpu.emit_pipeline(
            body,
            grid=(n // WINDOW,),
            in_specs=[pl.BlockSpec((1, WINDOW), index_map=lambda i: (0, i))],
            out_specs=[pl.BlockSpec((WINDOW, EMBED_DIM), index_map=lambda i: (i, 0))],
            core_axis_name=("core", "subcore"),
            dimension_semantics=(pltpu.PARALLEL,),
        )(idx_hbm, out_hbm)

    return k(table, idx)


def kernel(x, table):
    b, h = x.shape
    n = b * h
    idx = x.reshape(1, n).astype(jnp.int32)
    out = _sc_gather(table, idx, n)
    return out.reshape(b, h, EMBED_DIM)
